# baseline (speedup 1.0000x reference)
import functools

import jax
import jax.numpy as jnp
from jax import lax
from jax.experimental import pallas as pl
from jax.experimental.pallas import tpu as pltpu
from jax.experimental.pallas import tpu_sc as plsc

VOCAB = 100000
EMBED = 32
BATCH = 16384
FIELDS = 26
FPAD = 32

NW = 32
PER_W = BATCH // NW
NB = 32
NCHUNK = PER_W // NB

BB = 512
TC_GRID = BATCH // BB


def _gather_body(table_hbm, idx_hbm, val_hbm, out_hbm, idx_v, val_v, rows_v, sem):
    cid = lax.axis_index("c")
    sid = lax.axis_index("s")
    wid = sid * 2 + cid
    base = wid * PER_W

    def chunk_step(c, _):
        b0 = pl.multiple_of(base + c * NB, NB)
        pltpu.sync_copy(idx_hbm.at[pl.ds(b0, NB)], idx_v)
        pltpu.sync_copy(val_hbm.at[pl.ds(b0, NB)], val_v)
        copies = []
        for b in range(NB):
            copies.append(
                pltpu.async_copy(
                    table_hbm.at[idx_v.at[b]],
                    rows_v.at[b, pl.ds(0, FIELDS)],
                    sem,
                )
            )
        for cp in copies:
            cp.wait()

        def mul_row(b, _):
            v_lo = val_v[b, pl.ds(0, 16)]
            v_hi = val_v[b, pl.ds(10, 16)]
            for f in range(FIELDS):
                s = jnp.broadcast_to(v_lo[f] if f < 16 else v_hi[f - 10], (16,))
                rows_v[b, f, pl.ds(0, 16)] = rows_v[b, f, pl.ds(0, 16)] * s
                rows_v[b, f, pl.ds(16, 16)] = rows_v[b, f, pl.ds(16, 16)] * s
            return 0

        lax.fori_loop(0, NB, mul_row, 0)

        pltpu.sync_copy(rows_v, out_hbm.at[pl.ds(b0, NB)])
        return 0

    lax.fori_loop(0, NCHUNK, chunk_step, 0)


def _sc_gather_mul(table, idx, val):
    kern = functools.partial(
        pl.kernel,
        out_type=jax.ShapeDtypeStruct((BATCH, FPAD, EMBED), jnp.float32),
        mesh=plsc.VectorSubcoreMesh(core_axis_name="c", subcore_axis_name="s"),
        scratch_types=[
            pltpu.VMEM((NB, FIELDS), jnp.int32),
            pltpu.VMEM((NB, FIELDS), jnp.float32),
            pltpu.VMEM((NB, FPAD, EMBED), jnp.float32),
            pltpu.SemaphoreType.DMA,
        ],
        compiler_params=pltpu.CompilerParams(use_tc_tiling_on_sc=False),
    )(_gather_body)
    return kern(table, idx, val)


def _fmt_body(rows_ref, out_ref):
    out_ref[...] = rows_ref[:, :FIELDS, :]


def _tc_format(rows):
    return pl.pallas_call(
        _fmt_body,
        grid=(TC_GRID,),
        in_specs=[pl.BlockSpec((BB, FPAD, EMBED), lambda i: (i, 0, 0))],
        out_specs=pl.BlockSpec((BB, FIELDS, EMBED), lambda i: (i, 0, 0)),
        out_shape=jax.ShapeDtypeStruct((BATCH, FIELDS, EMBED), jnp.float32),
    )(rows)


def kernel(embed_index, embed_value, table):
    idx = embed_index.astype(jnp.int32)
    rows = _sc_gather_mul(table, idx, embed_value)
    return _tc_format(rows)


# speedup vs baseline: 2.0629x; 2.0629x over previous
import functools

import jax
import jax.numpy as jnp
from jax import lax
from jax.experimental import pallas as pl
from jax.experimental.pallas import tpu as pltpu
from jax.experimental.pallas import tpu_sc as plsc

VOCAB = 100000
EMBED = 32
BATCH = 16384
FIELDS = 26
FPAD = 28

NW = 32
PER_W = BATCH // NW
NB = 32
NCHUNK = PER_W // NB

BB = 512
TC_GRID = BATCH // BB
ROWS7 = BATCH * FPAD * EMBED // 128


def _gather_body(table_hbm, idx_hbm, out_hbm, idx_v, rows_v, sem):
    cid = lax.axis_index("c")
    sid = lax.axis_index("s")
    wid = sid * 2 + cid
    base = wid * PER_W

    def chunk_step(c, _):
        b0 = pl.multiple_of(base + c * NB, NB)
        pltpu.sync_copy(idx_hbm.at[pl.ds(b0, NB)], idx_v)
        copies = []
        for b in range(NB):
            copies.append(
                pltpu.async_copy(
                    table_hbm.at[idx_v.at[b]],
                    rows_v.at[b, pl.ds(0, FIELDS)],
                    sem,
                )
            )
        for cp in copies:
            cp.wait()
        pltpu.sync_copy(rows_v, out_hbm.at[pl.ds(b0, NB)])
        return 0

    lax.fori_loop(0, NCHUNK, chunk_step, 0)


def _sc_gather(table, idx):
    kern = functools.partial(
        pl.kernel,
        out_type=jax.ShapeDtypeStruct((BATCH, FPAD, EMBED), jnp.float32),
        mesh=plsc.VectorSubcoreMesh(core_axis_name="c", subcore_axis_name="s"),
        scratch_types=[
            pltpu.VMEM((NB, FIELDS), jnp.int32),
            pltpu.VMEM((NB, FPAD, EMBED), jnp.float32),
            pltpu.SemaphoreType.DMA,
        ],
        compiler_params=pltpu.CompilerParams(use_tc_tiling_on_sc=False),
    )(_gather_body)
    return kern(table, idx)


def _mul_body(rows_ref, val_ref, out_ref):
    x = rows_ref[...]
    x = x.reshape(BB * 7, 4, EMBED)
    x = x + 0.0
    x = x.reshape(BB, FPAD, EMBED)
    x = x + 0.0
    x = x[:, :FIELDS, :]
    xt = jnp.transpose(x, (1, 2, 0))
    v = val_ref[...]
    out_ref[...] = xt * v[:, None, :]


def _tc_mul_t(rows2d, val_t):
    return pl.pallas_call(
        _mul_body,
        grid=(TC_GRID,),
        in_specs=[
            pl.BlockSpec((BB * FPAD * EMBED // 128, 128), lambda i: (i, 0)),
            pl.BlockSpec((FIELDS, BB), lambda i: (0, i)),
        ],
        out_specs=pl.BlockSpec((FIELDS, EMBED, BB), lambda i: (0, 0, i)),
        out_shape=jax.ShapeDtypeStruct((FIELDS, EMBED, BATCH), jnp.float32),
    )(rows2d, val_t)


def kernel(embed_index, embed_value, table):
    idx = embed_index.astype(jnp.int32)
    rows = _sc_gather(table, idx)
    rows2d = rows.reshape(ROWS7, 128)
    out_t = _tc_mul_t(rows2d, embed_value.T)
    return jnp.transpose(out_t, (2, 0, 1))
